# Initial kernel scaffold; baseline (speedup 1.0000x reference)
import jax
import jax.numpy as jnp
from jax.experimental import pallas as pl


def kernel(news_ids, category_ids, news_table, category_table):
    raise NotImplementedError("write your pallas kernel here")


# speedup vs baseline: 1.5049x; 1.5049x over previous
import functools

import jax
import jax.numpy as jnp
from jax import lax
from jax.experimental import pallas as pl
from jax.experimental.pallas import tpu as pltpu
from jax.experimental.pallas import tpu_sc as plsc

BATCH = 4096
SEQ_LEN = 50
NEWS_DIM = 64
CAT_DIM = 16
OUT_DIM = NEWS_DIM + CAT_DIM

TOTAL = BATCH * SEQ_LEN
NUM_WORKERS = 32
PER_WORKER = TOTAL // NUM_WORKERS
CHUNK = 128
CHUNKS_PER_WORKER = PER_WORKER // CHUNK


def _make_body():
    mesh = plsc.VectorSubcoreMesh(core_axis_name="c", subcore_axis_name="s")
    return functools.partial(
        pl.kernel,
        mesh=mesh,
        out_type=jax.ShapeDtypeStruct((TOTAL, OUT_DIM), jnp.float32),
        scratch_types=[
            pltpu.VMEM((CHUNKS_PER_WORKER, CHUNK), jnp.int32),
            pltpu.VMEM((CHUNKS_PER_WORKER, CHUNK), jnp.int32),
            pltpu.VMEM((CHUNK, NEWS_DIM), jnp.float32),
            pltpu.VMEM((CHUNK, CAT_DIM), jnp.float32),
            pltpu.SemaphoreType.DMA,
            pltpu.SemaphoreType.DMA,
        ],
        compiler_params=pltpu.CompilerParams(use_tc_tiling_on_sc=False),
    )


@_make_body()
def _joint_embed(nidx_hbm, cidx_hbm, ntab_hbm, ctab_hbm, out_hbm,
                 nidx_v, cidx_v, news_v, cat_v, sem_n, sem_c):
    wid = lax.axis_index("s") * 2 + lax.axis_index("c")
    pltpu.sync_copy(nidx_hbm.at[wid], nidx_v)
    pltpu.sync_copy(cidx_hbm.at[wid], cidx_v)

    def step(j, carry):
        row0 = wid * PER_WORKER + j * CHUNK
        gn = pltpu.async_copy(ntab_hbm.at[nidx_v.at[j]], news_v, sem_n)
        gc = pltpu.async_copy(ctab_hbm.at[cidx_v.at[j]], cat_v, sem_c)
        gn.wait()
        gc.wait()
        pltpu.sync_copy(news_v, out_hbm.at[pl.ds(row0, CHUNK), pl.ds(0, NEWS_DIM)])
        pltpu.sync_copy(cat_v, out_hbm.at[pl.ds(row0, CHUNK), pl.ds(NEWS_DIM, CAT_DIM)])
        return carry

    lax.fori_loop(0, CHUNKS_PER_WORKER, step, 0)


def kernel(news_ids, category_ids, news_table, category_table):
    nidx = news_ids.reshape(NUM_WORKERS, CHUNKS_PER_WORKER, CHUNK)
    cidx = category_ids.reshape(NUM_WORKERS, CHUNKS_PER_WORKER, CHUNK)
    out = _joint_embed(nidx, cidx, news_table, category_table)
    return out.reshape(BATCH, SEQ_LEN, OUT_DIM)


# speedup vs baseline: 1.5382x; 1.0221x over previous
import functools

import jax
import jax.numpy as jnp
from jax import lax
from jax.experimental import pallas as pl
from jax.experimental.pallas import tpu as pltpu
from jax.experimental.pallas import tpu_sc as plsc

BATCH = 4096
SEQ_LEN = 50
NEWS_DIM = 64
CAT_DIM = 16
OUT_DIM = NEWS_DIM + CAT_DIM

TOTAL = BATCH * SEQ_LEN
NUM_WORKERS = 32
PER_WORKER = TOTAL // NUM_WORKERS
CHUNK = 128
CHUNKS_PER_WORKER = PER_WORKER // CHUNK
K = 5
SUPER = K * CHUNK
NSUPER = PER_WORKER // SUPER


def _make_body():
    mesh = plsc.VectorSubcoreMesh(core_axis_name="c", subcore_axis_name="s")
    return functools.partial(
        pl.kernel,
        mesh=mesh,
        out_type=jax.ShapeDtypeStruct((TOTAL, OUT_DIM), jnp.float32),
        scratch_types=[
            pltpu.VMEM((CHUNKS_PER_WORKER, CHUNK), jnp.int32),
            pltpu.VMEM((CHUNKS_PER_WORKER, CHUNK), jnp.int32),
            pltpu.VMEM((SUPER, NEWS_DIM), jnp.float32),
            pltpu.VMEM((SUPER, NEWS_DIM), jnp.float32),
            pltpu.VMEM((SUPER, CAT_DIM), jnp.float32),
            pltpu.VMEM((SUPER, CAT_DIM), jnp.float32),
            pltpu.SemaphoreType.DMA,
            pltpu.SemaphoreType.DMA,
            pltpu.SemaphoreType.DMA,
            pltpu.SemaphoreType.DMA,
        ],
        compiler_params=pltpu.CompilerParams(use_tc_tiling_on_sc=False),
    )


@_make_body()
def _joint_embed(nidx_hbm, cidx_hbm, ntab_hbm, ctab_hbm, out_hbm,
                 nidx_v, cidx_v, news_v0, news_v1, cat_v0, cat_v1,
                 sem_g0, sem_g1, sem_w0, sem_w1):
    wid = lax.axis_index("s") * 2 + lax.axis_index("c")
    news_v = (news_v0, news_v1)
    cat_v = (cat_v0, cat_v1)
    sem_g = (sem_g0, sem_g1)
    sem_w = (sem_w0, sem_w1)

    pltpu.sync_copy(nidx_hbm.at[wid], nidx_v)
    pltpu.sync_copy(cidx_hbm.at[wid], cidx_v)

    def issue_gathers(s, b):
        handles = []
        for k in range(K):
            j = s * K + k
            handles.append(pltpu.async_copy(
                ntab_hbm.at[nidx_v.at[j]],
                news_v[b].at[pl.ds(k * CHUNK, CHUNK)], sem_g[b]))
            handles.append(pltpu.async_copy(
                ctab_hbm.at[cidx_v.at[j]],
                cat_v[b].at[pl.ds(k * CHUNK, CHUNK)], sem_g[b]))
        return handles

    def issue_writes(s, b):
        row0 = wid * PER_WORKER + s * SUPER
        return [
            pltpu.async_copy(
                news_v[b], out_hbm.at[pl.ds(row0, SUPER), pl.ds(0, NEWS_DIM)],
                sem_w[b]),
            pltpu.async_copy(
                cat_v[b], out_hbm.at[pl.ds(row0, SUPER), pl.ds(NEWS_DIM, CAT_DIM)],
                sem_w[b]),
        ]

    gather_handles = {0: issue_gathers(0, 0)}
    write_handles = {}
    for s in range(NSUPER):
        b = s % 2
        if s + 1 < NSUPER:
            if s - 1 >= 0:
                for h in write_handles.pop(s - 1):
                    h.wait()
            gather_handles[s + 1] = issue_gathers(s + 1, 1 - b)
        for h in gather_handles.pop(s):
            h.wait()
        write_handles[s] = issue_writes(s, b)
    for s, hs in sorted(write_handles.items()):
        for h in hs:
            h.wait()


def kernel(news_ids, category_ids, news_table, category_table):
    nidx = news_ids.reshape(NUM_WORKERS, CHUNKS_PER_WORKER, CHUNK)
    cidx = category_ids.reshape(NUM_WORKERS, CHUNKS_PER_WORKER, CHUNK)
    out = _joint_embed(nidx, cidx, news_table, category_table)
    return out.reshape(BATCH, SEQ_LEN, OUT_DIM)


# speedup vs baseline: 1.5425x; 1.0028x over previous
import functools

import jax
import jax.numpy as jnp
from jax import lax
from jax.experimental import pallas as pl
from jax.experimental.pallas import tpu as pltpu
from jax.experimental.pallas import tpu_sc as plsc

BATCH = 4096
SEQ_LEN = 50
NEWS_DIM = 64
CAT_DIM = 16
OUT_DIM = NEWS_DIM + CAT_DIM

TOTAL = BATCH * SEQ_LEN
NUM_WORKERS = 32
PER_WORKER = TOTAL // NUM_WORKERS
CHUNK = 640
CHUNKS_PER_WORKER = PER_WORKER // CHUNK
K = 1
SUPER = K * CHUNK
NSUPER = PER_WORKER // SUPER


def _make_body():
    mesh = plsc.VectorSubcoreMesh(core_axis_name="c", subcore_axis_name="s")
    return functools.partial(
        pl.kernel,
        mesh=mesh,
        out_type=jax.ShapeDtypeStruct((TOTAL, OUT_DIM), jnp.float32),
        scratch_types=[
            pltpu.VMEM((CHUNKS_PER_WORKER, CHUNK), jnp.int32),
            pltpu.VMEM((CHUNKS_PER_WORKER, CHUNK), jnp.int32),
            pltpu.VMEM((SUPER, NEWS_DIM), jnp.float32),
            pltpu.VMEM((SUPER, NEWS_DIM), jnp.float32),
            pltpu.VMEM((SUPER, CAT_DIM), jnp.float32),
            pltpu.VMEM((SUPER, CAT_DIM), jnp.float32),
            pltpu.SemaphoreType.DMA,
            pltpu.SemaphoreType.DMA,
            pltpu.SemaphoreType.DMA,
            pltpu.SemaphoreType.DMA,
        ],
        compiler_params=pltpu.CompilerParams(use_tc_tiling_on_sc=False),
    )


@_make_body()
def _joint_embed(nidx_hbm, cidx_hbm, ntab_hbm, ctab_hbm, out_hbm,
                 nidx_v, cidx_v, news_v0, news_v1, cat_v0, cat_v1,
                 sem_g0, sem_g1, sem_w0, sem_w1):
    wid = lax.axis_index("s") * 2 + lax.axis_index("c")
    news_v = (news_v0, news_v1)
    cat_v = (cat_v0, cat_v1)
    sem_g = (sem_g0, sem_g1)
    sem_w = (sem_w0, sem_w1)

    pltpu.sync_copy(nidx_hbm.at[wid], nidx_v)
    pltpu.sync_copy(cidx_hbm.at[wid], cidx_v)

    def issue_gathers(s, b):
        handles = []
        for k in range(K):
            j = s * K + k
            handles.append(pltpu.async_copy(
                ntab_hbm.at[nidx_v.at[j]],
                news_v[b].at[pl.ds(k * CHUNK, CHUNK)], sem_g[b]))
            handles.append(pltpu.async_copy(
                ctab_hbm.at[cidx_v.at[j]],
                cat_v[b].at[pl.ds(k * CHUNK, CHUNK)], sem_g[b]))
        return handles

    def issue_writes(s, b):
        row0 = wid * PER_WORKER + s * SUPER
        return [
            pltpu.async_copy(
                news_v[b], out_hbm.at[pl.ds(row0, SUPER), pl.ds(0, NEWS_DIM)],
                sem_w[b]),
            pltpu.async_copy(
                cat_v[b], out_hbm.at[pl.ds(row0, SUPER), pl.ds(NEWS_DIM, CAT_DIM)],
                sem_w[b]),
        ]

    gather_handles = {0: issue_gathers(0, 0)}
    write_handles = {}
    for s in range(NSUPER):
        b = s % 2
        if s + 1 < NSUPER:
            if s - 1 >= 0:
                for h in write_handles.pop(s - 1):
                    h.wait()
            gather_handles[s + 1] = issue_gathers(s + 1, 1 - b)
        for h in gather_handles.pop(s):
            h.wait()
        write_handles[s] = issue_writes(s, b)
    for s, hs in sorted(write_handles.items()):
        for h in hs:
            h.wait()


def kernel(news_ids, category_ids, news_table, category_table):
    nidx = news_ids.reshape(NUM_WORKERS, CHUNKS_PER_WORKER, CHUNK)
    cidx = category_ids.reshape(NUM_WORKERS, CHUNKS_PER_WORKER, CHUNK)
    out = _joint_embed(nidx, cidx, news_table, category_table)
    return out.reshape(BATCH, SEQ_LEN, OUT_DIM)


# speedup vs baseline: 1.5668x; 1.0157x over previous
import functools

import jax
import jax.numpy as jnp
from jax import lax
from jax.experimental import pallas as pl
from jax.experimental.pallas import tpu as pltpu
from jax.experimental.pallas import tpu_sc as plsc

BATCH = 4096
SEQ_LEN = 50
NEWS_DIM = 64
CAT_DIM = 16
OUT_DIM = NEWS_DIM + CAT_DIM

TOTAL = BATCH * SEQ_LEN
NUM_WORKERS = 32
PER_WORKER = TOTAL // NUM_WORKERS
CHUNK = 128
CHUNKS_PER_WORKER = PER_WORKER // CHUNK
K = 5
SUPER = K * CHUNK
NSUPER = PER_WORKER // SUPER
GROUPS = SUPER // 16


def _make_body():
    mesh = plsc.VectorSubcoreMesh(core_axis_name="c", subcore_axis_name="s")
    return functools.partial(
        pl.kernel,
        mesh=mesh,
        out_type=jax.ShapeDtypeStruct((TOTAL, OUT_DIM), jnp.float32),
        scratch_types=[
            pltpu.VMEM((CHUNKS_PER_WORKER, CHUNK), jnp.int32),
            pltpu.VMEM((PER_WORKER,), jnp.int32),
            pltpu.VMEM((1000 * CAT_DIM,), jnp.float32),
            pltpu.VMEM((SUPER, NEWS_DIM), jnp.float32),
            pltpu.VMEM((SUPER, NEWS_DIM), jnp.float32),
            pltpu.VMEM((SUPER, CAT_DIM), jnp.float32),
            pltpu.SemaphoreType.DMA,
            pltpu.SemaphoreType.DMA,
            pltpu.SemaphoreType.DMA,
            pltpu.SemaphoreType.DMA,
            pltpu.SemaphoreType.DMA,
        ],
        compiler_params=pltpu.CompilerParams(
            use_tc_tiling_on_sc=False, needs_layout_passes=False),
    )


@_make_body()
def _joint_embed(nidx_hbm, cidx_hbm, ntab_hbm, ctab_hbm, out_hbm,
                 nidx_v, cidx_v, ctab_v, news_v0, news_v1, cat_v,
                 sem_g0, sem_g1, sem_wn0, sem_wn1, sem_wc):
    wid = lax.axis_index("s") * 2 + lax.axis_index("c")
    news_v = (news_v0, news_v1)
    sem_g = (sem_g0, sem_g1)
    sem_wn = (sem_wn0, sem_wn1)

    pltpu.sync_copy(nidx_hbm.at[wid], nidx_v)
    pltpu.sync_copy(cidx_hbm.at[wid], cidx_v)
    pltpu.sync_copy(ctab_hbm, ctab_v)

    iota16 = lax.iota(jnp.int32, 16)
    dnums = lax.GatherDimensionNumbers(
        offset_dims=(), collapsed_slice_dims=(0,), start_index_map=(0,))

    def bcast_lane(vec, l):
        idx = jnp.full((16, 1), l, jnp.int32)
        return lax.gather(vec, idx, dnums, slice_sizes=(1,),
                          mode=lax.GatherScatterMode.PROMISE_IN_BOUNDS)

    def issue_gathers(s, b):
        return [
            pltpu.async_copy(
                ntab_hbm.at[nidx_v.at[s * K + k]],
                news_v[b].at[pl.ds(k * CHUNK, CHUNK)], sem_g[b])
            for k in range(K)
        ]

    def compute_cat(s):
        def group(g, carry):
            c16 = cidx_v[pl.ds(s * SUPER + g * 16, 16)]
            for l in range(16):
                cb = bcast_lane(c16, l)
                row = plsc.load_gather(ctab_v, [cb * CAT_DIM + iota16])
                cat_v[g * 16 + l, :] = row
            return carry
        lax.fori_loop(0, GROUPS, group, 0)

    def issue_news_write(s, b):
        row0 = wid * PER_WORKER + s * SUPER
        return pltpu.async_copy(
            news_v[b], out_hbm.at[pl.ds(row0, SUPER), pl.ds(0, NEWS_DIM)],
            sem_wn[b])

    def issue_cat_write(s):
        row0 = wid * PER_WORKER + s * SUPER
        return pltpu.async_copy(
            cat_v, out_hbm.at[pl.ds(row0, SUPER), pl.ds(NEWS_DIM, CAT_DIM)],
            sem_wc)

    gather_handles = {0: issue_gathers(0, 0)}
    news_writes = {}
    cat_write = None
    for s in range(NSUPER):
        b = s % 2
        if s + 1 < NSUPER:
            if s - 1 >= 0:
                news_writes.pop(s - 1).wait()
            gather_handles[s + 1] = issue_gathers(s + 1, 1 - b)
        if cat_write is not None:
            cat_write.wait()
        compute_cat(s)
        for h in gather_handles.pop(s):
            h.wait()
        news_writes[s] = issue_news_write(s, b)
        cat_write = issue_cat_write(s)
    for s in sorted(news_writes):
        news_writes.pop(s).wait()
    cat_write.wait()


def kernel(news_ids, category_ids, news_table, category_table):
    nidx = news_ids.reshape(NUM_WORKERS, CHUNKS_PER_WORKER, CHUNK)
    cidx = category_ids.reshape(NUM_WORKERS, PER_WORKER)
    ctab = category_table.reshape(1000 * CAT_DIM)
    out = _joint_embed(nidx, cidx, news_table, ctab)
    return out.reshape(BATCH, SEQ_LEN, OUT_DIM)
